# baseline (speedup 1.0000x reference)
import functools

import jax
import jax.numpy as jnp
from jax import lax
from jax.experimental import pallas as pl
from jax.experimental.pallas import tpu as pltpu
from jax.experimental.pallas import tpu_sc as plsc

N = 10000
D = 128
H = 128
C = 32
G = 128

NC = 2
NS = 16
NW = NC * NS
L = 16

K = 128
NB = 3
NE = 4
NCH0 = 155


def _lane_splat(vec, lane):
    idx = jnp.full((L, 1), lane, jnp.int32)
    dn = lax.GatherDimensionNumbers(
        offset_dims=(), collapsed_slice_dims=(0,), start_index_map=(0,))
    return lax.gather(vec, idx, dn, (1,),
                      mode=lax.GatherScatterMode.PROMISE_IN_BOUNDS)


def _spmm_body(h_hbm, src_hbm, dst_hbm, w_hbm, out_hbm,
               agg_sh, rows, sring, dring, wring, gsem, ssem, esem):
    cid = lax.axis_index("c")
    sid = lax.axis_index("s")
    total_chunks = src_hbm.shape[0] // (K * NS)
    nchunk = jnp.where(cid == 0, NCH0, total_chunks - NCH0)
    ebase = (cid * NS * NCH0 + sid * nchunk) * K

    def e_start(c, t):
        base = ebase + c * K
        pltpu.make_async_copy(src_hbm.at[pl.ds(base, K)], sring.at[t],
                              esem).start()
        pltpu.make_async_copy(dst_hbm.at[pl.ds(base, K)], dring.at[t],
                              esem).start()
        pltpu.make_async_copy(w_hbm.at[pl.ds(base, K)], wring.at[t],
                              esem).start()

    def e_wait(t):
        pltpu.make_async_copy(src_hbm.at[pl.ds(0, K)], sring.at[t],
                              esem).wait()
        pltpu.make_async_copy(dst_hbm.at[pl.ds(0, K)], dring.at[t],
                              esem).wait()
        pltpu.make_async_copy(w_hbm.at[pl.ds(0, K)], wring.at[t],
                              esem).wait()

    def g_desc(t, b):
        return pltpu.make_async_copy(h_hbm.at[sring.at[t]], rows.at[b], gsem)

    def s_desc(t, b):
        return pltpu.make_async_copy(rows.at[b], agg_sh.at[dring.at[t]],
                                     ssem)

    for c in range(NB):
        e_start(c, c)
    e_wait(0)
    g_desc(0, 0).start()
    e_wait(1)
    g_desc(1, 1).start()

    def _zrow(r, carry):
        for k in range(H // L):
            rows[2, r, pl.ds(k * L, L)] = jnp.zeros((L,), jnp.float32)
        return carry

    lax.fori_loop(0, K, _zrow, 0)

    def _regions(fn):
        @pl.when(sid < NS - 1)
        def _full():
            for z in range(5):
                fn(pl.ds(sid * 640 + z * 128, 128), 128)

        @pl.when(sid == NS - 1)
        def _tail():
            for z in range(5):
                fn(pl.ds((NS - 1) * 640 + z * 80, 80), 80)

    _regions(lambda sl, nr: pltpu.sync_copy(
        rows.at[2].at[pl.ds(0, nr)], agg_sh.at[sl]))
    plsc.subcore_barrier()

    def _step(c, carry):
        b = lax.rem(c, NB)
        t = lax.rem(c, NE)
        b1 = lax.rem(c + 2, NB)
        t2 = lax.rem(c + 2, NE)
        t3 = lax.rem(c + 3, NE)
        g_desc(t, b).wait()

        def _scale(bs):
            def _group(g, gcarry):
                wvec = wring[t, pl.ds(g * L, L)]
                ws = [_lane_splat(wvec, lane) for lane in range(L)]
                for lane in range(L):
                    e = g * L + lane
                    for k in range(H // L):
                        sl = pl.ds(k * L, L)
                        rows[bs, e, sl] = rows[bs, e, sl] * ws[lane]
                return gcarry

            lax.fori_loop(0, K // L, _group, 0)

        for bs in range(NB):
            @pl.when(b == bs)
            def _sc(bs=bs):
                _scale(bs)

        s_desc(t, b).start(add=True)

        @pl.when(c > 0)
        def _wprev():
            s_desc(t3, b1).wait()

        @pl.when(c + 2 < nchunk)
        def _gnext():
            e_wait(t2)
            g_desc(t2, b1).start()

        @pl.when(c + 3 < nchunk)
        def _enext():
            e_start(c + 3, t3)

        return carry

    lax.fori_loop(0, nchunk, _step, 0)
    s_desc(lax.rem(nchunk - 1, NE), lax.rem(nchunk - 1, NB)).wait()
    plsc.subcore_barrier()

    _regions(lambda sl, nr: pltpu.sync_copy(
        agg_sh.at[sl], out_hbm.at[cid].at[sl]))


def _make_spmm():
    mesh = plsc.VectorSubcoreMesh(
        core_axis_name="c", subcore_axis_name="s",
        num_cores=NC, num_subcores=NS)
    return pl.kernel(
        _spmm_body,
        out_type=jax.ShapeDtypeStruct((NC, N, H), jnp.float32),
        mesh=mesh,
        scratch_types=[
            pltpu.VMEM_SHARED((N, H), jnp.float32),
            pltpu.VMEM((NB, K, H), jnp.float32),
            pltpu.VMEM((NE, K), jnp.int32),
            pltpu.VMEM((NE, K), jnp.int32),
            pltpu.VMEM((NE, K), jnp.float32),
        ] + [pltpu.SemaphoreType.DMA] * 3,
    )


_spmm = _make_spmm()


def _lin2_body(x_ref, wr_ref, wn_ref, hr_ref, hn_ref):
    x = x_ref[...]
    hr_ref[...] = jnp.dot(x, wr_ref[...], preferred_element_type=jnp.float32)
    hn_ref[...] = jnp.dot(x, wn_ref[...], preferred_element_type=jnp.float32)


_lin2 = pl.pallas_call(
    _lin2_body,
    out_shape=(jax.ShapeDtypeStruct((N, H), jnp.float32),
               jax.ShapeDtypeStruct((N, H), jnp.float32)),
)


def _bn_relu(hr, agg, b, gamma, beta):
    t = hr + agg[0] + agg[1] + b
    m = jnp.mean(t, axis=0)
    v = jnp.var(t, axis=0)
    h = (t - m) / jnp.sqrt(v + 1e-5) * gamma + beta
    return jnp.maximum(h, 0.0)


def _bnlin_body(hr_ref, agg_ref, b_ref, g_ref, be_ref, wr_ref, wn_ref,
                hr2_ref, hn2_ref):
    h = _bn_relu(hr_ref[...], agg_ref[...], b_ref[...], g_ref[...], be_ref[...])
    hr2_ref[...] = jnp.dot(h, wr_ref[...], preferred_element_type=jnp.float32)
    hn2_ref[...] = jnp.dot(h, wn_ref[...], preferred_element_type=jnp.float32)


_bnlin = pl.pallas_call(
    _bnlin_body,
    out_shape=(jax.ShapeDtypeStruct((N, H), jnp.float32),
               jax.ShapeDtypeStruct((N, H), jnp.float32)),
)


def _final_body(hr_ref, agg_ref, b_ref, g_ref, be_ref, batch_ref,
                wfc_ref, bfc_ref, wout_ref, bout_ref, out_ref):
    h = _bn_relu(hr_ref[...], agg_ref[...], b_ref[...], g_ref[...], be_ref[...])
    iot = lax.broadcasted_iota(jnp.int32, (N, G), 1)
    oh = jnp.where(batch_ref[...] == iot, 1.0, 0.0)
    pooled = lax.dot_general(oh, h, (((0,), (0,)), ((), ())),
                             preferred_element_type=jnp.float32)
    z = jnp.dot(pooled, wfc_ref[...], preferred_element_type=jnp.float32)
    z = z + bfc_ref[...]
    z = jnp.dot(z, wout_ref[...], preferred_element_type=jnp.float32)
    z = z + bout_ref[...]
    mz = jnp.max(z, axis=1, keepdims=True)
    lse = mz + jnp.log(jnp.sum(jnp.exp(z - mz), axis=1, keepdims=True))
    out_ref[...] = z - lse


_final = pl.pallas_call(
    _final_body,
    out_shape=jax.ShapeDtypeStruct((G, C), jnp.float32),
)


def kernel(x, edge_index, edge_attr, batch,
           W1_root, W1_nbr, b1, gamma1, beta1,
           W2_root, W2_nbr, b2, gamma2, beta2,
           W3_root, W3_nbr, b3,
           W_fc, b_fc, W_out, b_out):
    edge_index = edge_index.reshape(2, -1).astype(jnp.int32)
    e = edge_index.shape[1]
    unit = NW * K
    e_pad = ((e + unit - 1) // unit) * unit
    pad = e_pad - e
    src = jnp.concatenate([edge_index[0], jnp.zeros((pad,), jnp.int32)])
    dst = jnp.concatenate([edge_index[1], jnp.zeros((pad,), jnp.int32)])
    w = jnp.concatenate([edge_attr.reshape(-1).astype(jnp.float32),
                         jnp.zeros((pad,), jnp.float32)])
    batch_i = batch.astype(jnp.int32).reshape(N, 1)

    hr, hn = _lin2(x, W1_root, W1_nbr)
    agg = _spmm(hn, src, dst, w)
    hr, hn = _bnlin(hr, agg, b1, gamma1, beta1, W2_root, W2_nbr)
    agg = _spmm(hn, src, dst, w)
    hr, hn = _bnlin(hr, agg, b2, gamma2, beta2, W3_root, W3_nbr)
    agg = _spmm(hn, src, dst, w)
    return _final(hr, agg, b3, gamma2, beta2, batch_i,
                  W_fc, b_fc, W_out, b_out)


# speedup vs baseline: 1.1691x; 1.1691x over previous
import functools

import jax
import jax.numpy as jnp
from jax import lax
from jax.experimental import pallas as pl
from jax.experimental.pallas import tpu as pltpu
from jax.experimental.pallas import tpu_sc as plsc

N = 10000
D = 128
H = 128
C = 32
G = 128

NC = 2
NS = 16
NW = NC * NS
L = 16

K = 128
NB = 3
NE = 4
NCH0 = 139


def _lane_splat(vec, lane):
    idx = jnp.full((L, 1), lane, jnp.int32)
    dn = lax.GatherDimensionNumbers(
        offset_dims=(), collapsed_slice_dims=(0,), start_index_map=(0,))
    return lax.gather(vec, idx, dn, (1,),
                      mode=lax.GatherScatterMode.PROMISE_IN_BOUNDS)


def _spmm_body(h_hbm, src_hbm, dst_hbm, w_hbm, out_hbm,
               agg_sh, rows, sring, dring, wring, gsem, ssem, esem):
    cid = lax.axis_index("c")
    sid = lax.axis_index("s")
    total_chunks = src_hbm.shape[0] // (K * NS)
    nchunk = jnp.where(cid == 0, NCH0, total_chunks - NCH0)
    ebase = (cid * NS * NCH0 + sid * nchunk) * K

    def e_start(c, t):
        base = ebase + c * K
        pltpu.make_async_copy(src_hbm.at[pl.ds(base, K)], sring.at[t],
                              esem).start()
        pltpu.make_async_copy(dst_hbm.at[pl.ds(base, K)], dring.at[t],
                              esem).start()
        pltpu.make_async_copy(w_hbm.at[pl.ds(base, K)], wring.at[t],
                              esem).start()

    def e_wait(t):
        pltpu.make_async_copy(src_hbm.at[pl.ds(0, K)], sring.at[t],
                              esem).wait()
        pltpu.make_async_copy(dst_hbm.at[pl.ds(0, K)], dring.at[t],
                              esem).wait()
        pltpu.make_async_copy(w_hbm.at[pl.ds(0, K)], wring.at[t],
                              esem).wait()

    def g_desc(t, b):
        return pltpu.make_async_copy(h_hbm.at[sring.at[t]], rows.at[b], gsem)

    def s_desc(t, b):
        return pltpu.make_async_copy(rows.at[b], agg_sh.at[dring.at[t]],
                                     ssem)

    for c in range(NB):
        e_start(c, c)
    e_wait(0)
    g_desc(0, 0).start()
    e_wait(1)
    g_desc(1, 1).start()

    def _zrow(r, carry):
        for k in range(H // L):
            rows[2, r, pl.ds(k * L, L)] = jnp.zeros((L,), jnp.float32)
        return carry

    lax.fori_loop(0, K, _zrow, 0)

    def _regions(fn):
        @pl.when(sid < NS - 1)
        def _full():
            for z in range(5):
                fn(pl.ds(sid * 640 + z * 128, 128), 128)

        @pl.when(sid == NS - 1)
        def _tail():
            for z in range(5):
                fn(pl.ds((NS - 1) * 640 + z * 80, 80), 80)

    _regions(lambda sl, nr: pltpu.sync_copy(
        rows.at[2].at[pl.ds(0, nr)], agg_sh.at[sl]))
    plsc.subcore_barrier()

    def _step(c, carry):
        b = lax.rem(c, NB)
        t = lax.rem(c, NE)
        b1 = lax.rem(c + 2, NB)
        t2 = lax.rem(c + 2, NE)
        t3 = lax.rem(c + 3, NE)
        g_desc(t, b).wait()

        def _scale(bs):
            def _group(g, gcarry):
                wvec = wring[t, pl.ds(g * L, L)]
                ws = [_lane_splat(wvec, lane) for lane in range(L)]
                for lane in range(L):
                    e = g * L + lane
                    for k in range(H // L):
                        sl = pl.ds(k * L, L)
                        rows[bs, e, sl] = rows[bs, e, sl] * ws[lane]
                return gcarry

            lax.fori_loop(0, K // L, _group, 0)

        for bs in range(NB):
            @pl.when(b == bs)
            def _sc(bs=bs):
                _scale(bs)

        s_desc(t, b).start(add=True)

        @pl.when(c > 0)
        def _wprev():
            s_desc(t3, b1).wait()

        @pl.when(c + 2 < nchunk)
        def _gnext():
            e_wait(t2)
            g_desc(t2, b1).start()

        @pl.when(c + 3 < nchunk)
        def _enext():
            e_start(c + 3, t3)

        return carry

    lax.fori_loop(0, nchunk, _step, 0)
    s_desc(lax.rem(nchunk - 1, NE), lax.rem(nchunk - 1, NB)).wait()
    plsc.subcore_barrier()

    _regions(lambda sl, nr: pltpu.sync_copy(
        agg_sh.at[sl], out_hbm.at[cid].at[sl]))


def _make_spmm():
    mesh = plsc.VectorSubcoreMesh(
        core_axis_name="c", subcore_axis_name="s",
        num_cores=NC, num_subcores=NS)
    return pl.kernel(
        _spmm_body,
        out_type=jax.ShapeDtypeStruct((NC, N, H), jnp.float32),
        mesh=mesh,
        scratch_types=[
            pltpu.VMEM_SHARED((N, H), jnp.float32),
            pltpu.VMEM((NB, K, H), jnp.float32),
            pltpu.VMEM((NE, K), jnp.int32),
            pltpu.VMEM((NE, K), jnp.int32),
            pltpu.VMEM((NE, K), jnp.float32),
        ] + [pltpu.SemaphoreType.DMA] * 3,
    )


_spmm = _make_spmm()


def _lin2_body(x_ref, wr_ref, wn_ref, hr_ref, hn_ref):
    x = x_ref[...]
    hr_ref[...] = jnp.dot(x, wr_ref[...], preferred_element_type=jnp.float32)
    hn_ref[...] = jnp.dot(x, wn_ref[...], preferred_element_type=jnp.float32)


_lin2 = pl.pallas_call(
    _lin2_body,
    out_shape=(jax.ShapeDtypeStruct((N, H), jnp.float32),
               jax.ShapeDtypeStruct((N, H), jnp.float32)),
)


def _bn_relu(hr, agg, b, gamma, beta):
    t = hr + agg[0] + agg[1] + b
    m = jnp.mean(t, axis=0)
    v = jnp.var(t, axis=0)
    h = (t - m) / jnp.sqrt(v + 1e-5) * gamma + beta
    return jnp.maximum(h, 0.0)


def _bnlin_body(hr_ref, agg_ref, b_ref, g_ref, be_ref, wr_ref, wn_ref,
                hr2_ref, hn2_ref):
    h = _bn_relu(hr_ref[...], agg_ref[...], b_ref[...], g_ref[...], be_ref[...])
    hr2_ref[...] = jnp.dot(h, wr_ref[...], preferred_element_type=jnp.float32)
    hn2_ref[...] = jnp.dot(h, wn_ref[...], preferred_element_type=jnp.float32)


_bnlin = pl.pallas_call(
    _bnlin_body,
    out_shape=(jax.ShapeDtypeStruct((N, H), jnp.float32),
               jax.ShapeDtypeStruct((N, H), jnp.float32)),
)


def _final_body(hr_ref, agg_ref, b_ref, g_ref, be_ref, batch_ref,
                wfc_ref, bfc_ref, wout_ref, bout_ref, out_ref):
    h = _bn_relu(hr_ref[...], agg_ref[...], b_ref[...], g_ref[...], be_ref[...])
    iot = lax.broadcasted_iota(jnp.int32, (N, G), 1)
    oh = jnp.where(batch_ref[...] == iot, 1.0, 0.0)
    pooled = lax.dot_general(oh, h, (((0,), (0,)), ((), ())),
                             preferred_element_type=jnp.float32)
    z = jnp.dot(pooled, wfc_ref[...], preferred_element_type=jnp.float32)
    z = z + bfc_ref[...]
    z = jnp.dot(z, wout_ref[...], preferred_element_type=jnp.float32)
    z = z + bout_ref[...]
    mz = jnp.max(z, axis=1, keepdims=True)
    lse = mz + jnp.log(jnp.sum(jnp.exp(z - mz), axis=1, keepdims=True))
    out_ref[...] = z - lse


_final = pl.pallas_call(
    _final_body,
    out_shape=jax.ShapeDtypeStruct((G, C), jnp.float32),
)


def kernel(x, edge_index, edge_attr, batch,
           W1_root, W1_nbr, b1, gamma1, beta1,
           W2_root, W2_nbr, b2, gamma2, beta2,
           W3_root, W3_nbr, b3,
           W_fc, b_fc, W_out, b_out):
    edge_index = edge_index.reshape(2, -1).astype(jnp.int32)
    e = edge_index.shape[1]
    unit = NW * K
    e_pad = ((e + unit - 1) // unit) * unit
    pad = e_pad - e
    src = jnp.concatenate([edge_index[0], jnp.zeros((pad,), jnp.int32)])
    dst = jnp.concatenate([edge_index[1], jnp.zeros((pad,), jnp.int32)])
    w = jnp.concatenate([edge_attr.reshape(-1).astype(jnp.float32),
                         jnp.zeros((pad,), jnp.float32)])
    batch_i = batch.astype(jnp.int32).reshape(N, 1)

    hr, hn = _lin2(x, W1_root, W1_nbr)
    agg = _spmm(hn, src, dst, w)
    hr, hn = _bnlin(hr, agg, b1, gamma1, beta1, W2_root, W2_nbr)
    agg = _spmm(hn, src, dst, w)
    hr, hn = _bnlin(hr, agg, b2, gamma2, beta2, W3_root, W3_nbr)
    agg = _spmm(hn, src, dst, w)
    return _final(hr, agg, b3, gamma2, beta2, batch_i,
                  W_fc, b_fc, W_out, b_out)


# speedup vs baseline: 1.1694x; 1.0003x over previous
import functools

import jax
import jax.numpy as jnp
from jax import lax
from jax.experimental import pallas as pl
from jax.experimental.pallas import tpu as pltpu
from jax.experimental.pallas import tpu_sc as plsc

N = 10000
D = 128
H = 128
C = 32
G = 128

NC = 2
NS = 16
NW = NC * NS
L = 16

K = 128
NB = 3
NE = 4
NCH0 = 139


def _lane_splat(vec, lane):
    idx = jnp.full((L, 1), lane, jnp.int32)
    dn = lax.GatherDimensionNumbers(
        offset_dims=(), collapsed_slice_dims=(0,), start_index_map=(0,))
    return lax.gather(vec, idx, dn, (1,),
                      mode=lax.GatherScatterMode.PROMISE_IN_BOUNDS)


def _spmm_body(h_hbm, src_hbm, dst_hbm, w_hbm, out_hbm,
               agg_sh, rows, sring, dring, wring, gsem, ssem, esem):
    cid = lax.axis_index("c")
    sid = lax.axis_index("s")
    total_chunks = src_hbm.shape[0] // (K * NS)
    nchunk = jnp.where(cid == 0, NCH0, total_chunks - NCH0)
    ebase = (cid * NS * NCH0 + sid * nchunk) * K

    def e_start(c, t):
        base = ebase + c * K
        pltpu.make_async_copy(src_hbm.at[pl.ds(base, K)], sring.at[t],
                              esem).start()
        pltpu.make_async_copy(dst_hbm.at[pl.ds(base, K)], dring.at[t],
                              esem).start()
        pltpu.make_async_copy(w_hbm.at[pl.ds(base, K)], wring.at[t],
                              esem).start()

    def e_wait(t):
        pltpu.make_async_copy(src_hbm.at[pl.ds(0, K)], sring.at[t],
                              esem).wait()
        pltpu.make_async_copy(dst_hbm.at[pl.ds(0, K)], dring.at[t],
                              esem).wait()
        pltpu.make_async_copy(w_hbm.at[pl.ds(0, K)], wring.at[t],
                              esem).wait()

    def g_desc(t, b):
        return pltpu.make_async_copy(h_hbm.at[sring.at[t]], rows.at[b], gsem)

    def s_desc(t, b):
        return pltpu.make_async_copy(rows.at[b], agg_sh.at[dring.at[t]],
                                     ssem)

    for c in range(NB):
        e_start(c, c)
    e_wait(0)
    g_desc(0, 0).start()
    e_wait(1)
    g_desc(1, 1).start()

    def _zrow(r, carry):
        for k in range(H // L):
            rows[2, r, pl.ds(k * L, L)] = jnp.zeros((L,), jnp.float32)
        return carry

    lax.fori_loop(0, K, _zrow, 0)

    def _regions(fn):
        @pl.when(sid < NS - 1)
        def _full():
            for z in range(5):
                fn(pl.ds(sid * 640 + z * 128, 128), 128)

        @pl.when(sid == NS - 1)
        def _tail():
            for z in range(5):
                fn(pl.ds((NS - 1) * 640 + z * 80, 80), 80)

    _regions(lambda sl, nr: pltpu.sync_copy(
        rows.at[2].at[pl.ds(0, nr)], agg_sh.at[sl]))
    plsc.subcore_barrier()

    def _step(c, carry):
        b = lax.rem(c, NB)
        t = lax.rem(c, NE)
        b1 = lax.rem(c + 2, NB)
        t2 = lax.rem(c + 2, NE)
        t3 = lax.rem(c + 3, NE)
        g_desc(t, b).wait()

        def _scale(bs):
            def _group(g, gcarry):
                wvec = wring[t, pl.ds(g * L, L)]
                ws = [_lane_splat(wvec, lane) for lane in range(L)]
                for lane in range(L):
                    e = g * L + lane
                    for k in range(H // L):
                        sl = pl.ds(k * L, L)
                        rows[bs, e, sl] = rows[bs, e, sl] * ws[lane]
                return gcarry

            lax.fori_loop(0, K // L, _group, 0)

        for bs in range(NB):
            @pl.when(b == bs)
            def _sc(bs=bs):
                _scale(bs)

        s_desc(t, b).start(add=True)

        @pl.when(c > 0)
        def _wprev():
            s_desc(t3, b1).wait()

        @pl.when(c + 2 < nchunk)
        def _gnext():
            e_wait(t2)
            g_desc(t2, b1).start()

        @pl.when(c + 3 < nchunk)
        def _enext():
            e_start(c + 3, t3)

        return carry

    lax.fori_loop(0, nchunk, _step, 0)
    s_desc(lax.rem(nchunk - 1, NE), lax.rem(nchunk - 1, NB)).wait()
    plsc.subcore_barrier()

    _regions(lambda sl, nr: pltpu.sync_copy(
        agg_sh.at[sl], out_hbm.at[cid].at[sl]))


def _make_spmm():
    mesh = plsc.VectorSubcoreMesh(
        core_axis_name="c", subcore_axis_name="s",
        num_cores=NC, num_subcores=NS)
    return pl.kernel(
        _spmm_body,
        out_type=jax.ShapeDtypeStruct((NC, N, H), jnp.float32),
        mesh=mesh,
        scratch_types=[
            pltpu.VMEM_SHARED((N, H), jnp.float32),
            pltpu.VMEM((NB, K, H), jnp.float32),
            pltpu.VMEM((NE, K), jnp.int32),
            pltpu.VMEM((NE, K), jnp.int32),
            pltpu.VMEM((NE, K), jnp.float32),
        ] + [pltpu.SemaphoreType.DMA] * 3,
    )


_spmm_cache = []


def _get_spmm():
    if not _spmm_cache:
        _spmm_cache.append(_make_spmm())
    return _spmm_cache[0]


def _lin2_body(x_ref, wr_ref, wn_ref, hr_ref, hn_ref):
    x = x_ref[...]
    hr_ref[...] = jnp.dot(x, wr_ref[...], preferred_element_type=jnp.float32)
    hn_ref[...] = jnp.dot(x, wn_ref[...], preferred_element_type=jnp.float32)


_lin2 = pl.pallas_call(
    _lin2_body,
    out_shape=(jax.ShapeDtypeStruct((N, H), jnp.float32),
               jax.ShapeDtypeStruct((N, H), jnp.float32)),
)


def _bn_relu(hr, agg, b, gamma, beta):
    t = hr + agg[0] + agg[1] + b
    m = jnp.mean(t, axis=0)
    v = jnp.var(t, axis=0)
    h = (t - m) / jnp.sqrt(v + 1e-5) * gamma + beta
    return jnp.maximum(h, 0.0)


def _bnlin_body(hr_ref, agg_ref, b_ref, g_ref, be_ref, wr_ref, wn_ref,
                hr2_ref, hn2_ref):
    h = _bn_relu(hr_ref[...], agg_ref[...], b_ref[...], g_ref[...], be_ref[...])
    hr2_ref[...] = jnp.dot(h, wr_ref[...], preferred_element_type=jnp.float32)
    hn2_ref[...] = jnp.dot(h, wn_ref[...], preferred_element_type=jnp.float32)


_bnlin = pl.pallas_call(
    _bnlin_body,
    out_shape=(jax.ShapeDtypeStruct((N, H), jnp.float32),
               jax.ShapeDtypeStruct((N, H), jnp.float32)),
)


def _final_body(hr_ref, agg_ref, b_ref, g_ref, be_ref, batch_ref,
                wfc_ref, bfc_ref, wout_ref, bout_ref, out_ref):
    h = _bn_relu(hr_ref[...], agg_ref[...], b_ref[...], g_ref[...], be_ref[...])
    iot = lax.broadcasted_iota(jnp.int32, (N, G), 1)
    oh = jnp.where(batch_ref[...] == iot, 1.0, 0.0)
    pooled = lax.dot_general(oh, h, (((0,), (0,)), ((), ())),
                             preferred_element_type=jnp.float32)
    z = jnp.dot(pooled, wfc_ref[...], preferred_element_type=jnp.float32)
    z = z + bfc_ref[...]
    z = jnp.dot(z, wout_ref[...], preferred_element_type=jnp.float32)
    z = z + bout_ref[...]
    mz = jnp.max(z, axis=1, keepdims=True)
    lse = mz + jnp.log(jnp.sum(jnp.exp(z - mz), axis=1, keepdims=True))
    out_ref[...] = z - lse


_final = pl.pallas_call(
    _final_body,
    out_shape=jax.ShapeDtypeStruct((G, C), jnp.float32),
)


def kernel(x, edge_index, edge_attr, batch,
           W1_root, W1_nbr, b1, gamma1, beta1,
           W2_root, W2_nbr, b2, gamma2, beta2,
           W3_root, W3_nbr, b3,
           W_fc, b_fc, W_out, b_out):
    edge_index = edge_index.reshape(2, -1).astype(jnp.int32)
    e = edge_index.shape[1]
    unit = NW * K
    e_pad = ((e + unit - 1) // unit) * unit
    pad = e_pad - e
    src = jnp.concatenate([edge_index[0], jnp.zeros((pad,), jnp.int32)])
    dst = jnp.concatenate([edge_index[1], jnp.zeros((pad,), jnp.int32)])
    w = jnp.concatenate([edge_attr.reshape(-1).astype(jnp.float32),
                         jnp.zeros((pad,), jnp.float32)])
    batch_i = batch.astype(jnp.int32).reshape(N, 1)

    _spmm = _get_spmm()
    hr, hn = _lin2(x, W1_root, W1_nbr)
    agg = _spmm(hn, src, dst, w)
    hr, hn = _bnlin(hr, agg, b1, gamma1, beta1, W2_root, W2_nbr)
    agg = _spmm(hn, src, dst, w)
    hr, hn = _bnlin(hr, agg, b2, gamma2, beta2, W3_root, W3_nbr)
    agg = _spmm(hn, src, dst, w)
    return _final(hr, agg, b3, gamma2, beta2, batch_i,
                  W_fc, b_fc, W_out, b_out)
